# baseline (speedup 1.0000x reference)
import functools

import jax
import jax.numpy as jnp
from jax import lax
from jax.experimental import pallas as pl
from jax.experimental.pallas import tpu as pltpu
from jax.experimental.pallas import tpu_sc as plsc

NC = 2
NS = 16
L = 16
RW = 128
NW = NC * NS


def _geometry(n, m):
    rows = -(-n // (NW * RW))
    m_pad = (m // (NS * L) + 1) * NS * L
    return rows, m_pad


def _make_k1(n, m, rows, m_pad):
    msl = m_pad // NS
    ch = rows * RW
    v_last = n - (NW - 1) * ch
    tail = ch - v_last
    pr = ((m_pad - m) // L) * L

    def body(idx_hbm, chg_hbm, psum_hbm, pcnt_hbm,
             ssum, scnt, idx_v, val_v, zf, zi, ones_v, sem_st, sem_sc):
        c = lax.axis_index("c")
        s = lax.axis_index("s")
        wid = s * NC + c
        base = wid * ch

        @pl.when(wid < NW - 1)
        def _():
            pltpu.async_copy(idx_hbm.at[pl.ds(base, ch)], idx_v, sem_st)
            pltpu.async_copy(chg_hbm.at[pl.ds(base, ch)], val_v, sem_st)

        @pl.when(wid == NW - 1)
        def _():
            pltpu.async_copy(idx_hbm.at[pl.ds(base, v_last)],
                             idx_v.at[pl.ds(0, v_last)], sem_st)
            pltpu.async_copy(chg_hbm.at[pl.ds(base, v_last)],
                             val_v.at[pl.ds(0, v_last)], sem_st)

        def zbody(i, carry):
            zf[pl.ds(i * L, L)] = jnp.zeros((L,), jnp.float32)
            zi[pl.ds(i * L, L)] = jnp.zeros((L,), jnp.int32)
            return carry

        lax.fori_loop(0, msl // L, zbody, 0)
        for i in range(RW // L):
            ones_v[pl.ds(i * L, L)] = jnp.ones((L,), jnp.int32)
        pltpu.sync_copy(zf, ssum.at[pl.ds(s * msl, msl)])
        pltpu.sync_copy(zi, scnt.at[pl.ds(s * msl, msl)])
        plsc.subcore_barrier()

        @pl.when(wid < NW - 1)
        def _():
            pltpu.make_async_copy(idx_hbm.at[pl.ds(base, ch)],
                                  idx_v, sem_st).wait()
            pltpu.make_async_copy(chg_hbm.at[pl.ds(base, ch)],
                                  val_v, sem_st).wait()

        @pl.when(wid == NW - 1)
        def _():
            pltpu.make_async_copy(idx_hbm.at[pl.ds(base, v_last)],
                                  idx_v.at[pl.ds(0, v_last)], sem_st).wait()
            pltpu.make_async_copy(chg_hbm.at[pl.ds(base, v_last)],
                                  val_v.at[pl.ds(0, v_last)], sem_st).wait()

            def tbody(i, carry):
                lanes = pl.ds(v_last + i * L, L)
                idx_v[lanes] = (m + lax.rem(i * L, pr)
                                + lax.iota(jnp.int32, L))
                val_v[lanes] = jnp.zeros((L,), jnp.float32)
                return carry

            lax.fori_loop(0, tail // L, tbody, 0)

        ck = 7 if rows % 7 == 0 else 1
        nck = rows // ck

        def sbody(cidx, carry):
            for r in range(ck):
                rr = pl.ds((cidx * ck + r) * RW, RW)
                pltpu.async_copy(val_v.at[rr], ssum.at[idx_v.at[rr]],
                                 sem_sc, add=True)
                pltpu.async_copy(ones_v, scnt.at[idx_v.at[rr]],
                                 sem_sc, add=True)

            @pl.when(cidx > 0)
            def _():
                pltpu.make_async_copy(chg_hbm.at[pl.ds(0, 2 * ck * RW)],
                                      val_v.at[pl.ds(0, 2 * ck * RW)],
                                      sem_sc).wait()

            return carry

        lax.fori_loop(0, nck, sbody, 0)
        pltpu.make_async_copy(chg_hbm.at[pl.ds(0, 2 * ck * RW)],
                              val_v.at[pl.ds(0, 2 * ck * RW)], sem_sc).wait()
        plsc.subcore_barrier()
        pltpu.sync_copy(ssum.at[pl.ds(s * msl, msl)], zf)
        pltpu.sync_copy(zf, psum_hbm.at[pl.ds(c * m_pad + s * msl, msl)])
        pltpu.sync_copy(scnt.at[pl.ds(s * msl, msl)], zi)
        pltpu.sync_copy(zi, pcnt_hbm.at[pl.ds(c * m_pad + s * msl, msl)])

    return pl.kernel(
        body,
        out_type=[jax.ShapeDtypeStruct((NC * m_pad,), jnp.float32),
                  jax.ShapeDtypeStruct((NC * m_pad,), jnp.int32)],
        mesh=plsc.VectorSubcoreMesh(core_axis_name="c", subcore_axis_name="s",
                                    num_cores=NC, num_subcores=NS),
        scratch_types=[
            pltpu.VMEM_SHARED((m_pad,), jnp.float32),
            pltpu.VMEM_SHARED((m_pad,), jnp.int32),
            pltpu.VMEM((ch,), jnp.int32),
            pltpu.VMEM((ch,), jnp.float32),
            pltpu.VMEM((msl,), jnp.float32),
            pltpu.VMEM((msl,), jnp.int32),
            pltpu.VMEM((RW,), jnp.int32),
            pltpu.SemaphoreType.DMA,
            pltpu.SemaphoreType.DMA,
        ],
    )


def _make_k2(n, m, rows, m_pad):
    msl = m_pad // NS
    ch = rows * RW
    v_last = n - (NW - 1) * ch
    tail = ch - v_last
    pr = ((m_pad - m) // L) * L
    m_tail = m - (NS - 1) * msl
    CW = 1024

    def body(idx_hbm, chg_hbm, psum_hbm, pcnt_hbm, tot_hbm, out_hbm,
             scorr, idx_v, val_v, win_v, s0v, s1v, n0v, n1v, totv, cbuf,
             sem_st):
        c = lax.axis_index("c")
        s = lax.axis_index("s")
        wid = s * NC + c
        base = wid * ch
        bm = s * msl

        @pl.when(wid < NW - 1)
        def _():
            pltpu.async_copy(idx_hbm.at[pl.ds(base, ch)], idx_v, sem_st)
            pltpu.async_copy(chg_hbm.at[pl.ds(base, ch)], val_v, sem_st)

        @pl.when(wid == NW - 1)
        def _():
            pltpu.async_copy(idx_hbm.at[pl.ds(base, v_last)],
                             idx_v.at[pl.ds(0, v_last)], sem_st)
            pltpu.async_copy(chg_hbm.at[pl.ds(base, v_last)],
                             val_v.at[pl.ds(0, v_last)], sem_st)

        pltpu.sync_copy(psum_hbm.at[pl.ds(bm, msl)], s0v)
        pltpu.sync_copy(psum_hbm.at[pl.ds(m_pad + bm, msl)], s1v)
        pltpu.sync_copy(pcnt_hbm.at[pl.ds(bm, msl)], n0v)
        pltpu.sync_copy(pcnt_hbm.at[pl.ds(m_pad + bm, msl)], n1v)

        @pl.when(s < NS - 1)
        def _():
            pltpu.sync_copy(tot_hbm.at[pl.ds(bm, msl)], totv)

        @pl.when(s == NS - 1)
        def _():
            pltpu.sync_copy(tot_hbm.at[pl.ds(bm, m_tail)],
                            totv.at[pl.ds(0, m_tail)])

        def cbody(i, carry):
            sl = pl.ds(i * L, L)
            seg_sum = s0v[sl] + s1v[sl]
            cnt = (n0v[sl] + n1v[sl]).astype(jnp.float32)
            cbuf[sl] = (totv[sl] - seg_sum) / cnt
            return carry

        lax.fori_loop(0, msl // L, cbody, 0)
        pltpu.sync_copy(cbuf, scorr.at[pl.ds(bm, msl)])
        plsc.subcore_barrier()

        @pl.when(wid < NW - 1)
        def _():
            pltpu.make_async_copy(idx_hbm.at[pl.ds(base, ch)],
                                  idx_v, sem_st).wait()
            pltpu.make_async_copy(chg_hbm.at[pl.ds(base, ch)],
                                  val_v, sem_st).wait()

        @pl.when(wid == NW - 1)
        def _():
            pltpu.make_async_copy(idx_hbm.at[pl.ds(base, v_last)],
                                  idx_v.at[pl.ds(0, v_last)], sem_st).wait()
            pltpu.make_async_copy(chg_hbm.at[pl.ds(base, v_last)],
                                  val_v.at[pl.ds(0, v_last)], sem_st).wait()

            def tbody(i, carry):
                lanes = pl.ds(v_last + i * L, L)
                idx_v[lanes] = (m + lax.rem(i * L, pr)
                                + lax.iota(jnp.int32, L))
                val_v[lanes] = jnp.zeros((L,), jnp.float32)
                return carry

            lax.fori_loop(0, tail // L, tbody, 0)

        ve = jnp.where(wid == NW - 1, v_last, ch)
        lo8 = (idx_v[pl.ds(0, L)][0] // 8) * 8
        hi = idx_v[pl.ds(ve - L, L)][L - 1]
        width = hi - lo8 + 1
        ncw = (width + CW - 1) // CW
        start_cap = ((m_pad - CW) // 8) * 8

        def wbody(k, carry):
            st = jnp.minimum(lo8 + k * CW, start_cap)
            pltpu.sync_copy(scorr.at[pl.ds(st, CW)],
                            win_v.at[pl.ds(st - lo8, CW)])
            return carry

        lax.fori_loop(0, ncw, wbody, 0)

        lo_v = jnp.full((L,), lo8, jnp.int32)

        def gbody(j, carry):
            for l in range(RW // L):
                sl = pl.ds(j * RW + l * L, L)
                off = idx_v[sl] - lo_v
                g = plsc.load_gather(win_v, [off])
                val_v[sl] = val_v[sl] + g
            return carry

        lax.fori_loop(0, rows, gbody, 0)

        @pl.when(wid < NW - 1)
        def _():
            pltpu.sync_copy(val_v, out_hbm.at[pl.ds(base, ch)])

        @pl.when(wid == NW - 1)
        def _():
            pltpu.sync_copy(val_v.at[pl.ds(0, v_last)],
                            out_hbm.at[pl.ds(base, v_last)])

    return pl.kernel(
        body,
        out_type=jax.ShapeDtypeStruct((n,), jnp.float32),
        mesh=plsc.VectorSubcoreMesh(core_axis_name="c", subcore_axis_name="s",
                                    num_cores=NC, num_subcores=NS),
        compiler_params=pltpu.CompilerParams(needs_layout_passes=False),
        scratch_types=[
            pltpu.VMEM_SHARED((m_pad,), jnp.float32),
            pltpu.VMEM((ch,), jnp.int32),
            pltpu.VMEM((ch,), jnp.float32),
            pltpu.VMEM((m_pad,), jnp.float32),
            pltpu.VMEM((msl,), jnp.float32),
            pltpu.VMEM((msl,), jnp.float32),
            pltpu.VMEM((msl,), jnp.int32),
            pltpu.VMEM((msl,), jnp.int32),
            pltpu.VMEM((msl,), jnp.float32),
            pltpu.VMEM((msl,), jnp.float32),
            pltpu.SemaphoreType.DMA,
        ],
    )


def kernel(per_atom_charge, per_system_total_charge, atomic_subsystem_indices):
    n = per_atom_charge.shape[0]
    m = per_system_total_charge.shape[0]
    rows, m_pad = _geometry(n, m)

    chg = per_atom_charge.reshape(-1).astype(jnp.float32)
    idx = atomic_subsystem_indices.astype(jnp.int32)
    tot = per_system_total_charge.reshape(-1).astype(jnp.float32)

    psum, pcnt = _make_k1(n, m, rows, m_pad)(idx, chg)
    out = _make_k2(n, m, rows, m_pad)(idx, chg, psum, pcnt, tot)
    return out[:, None]


# speedup vs baseline: 1.1605x; 1.1605x over previous
import functools

import jax
import jax.numpy as jnp
from jax import lax
from jax.experimental import pallas as pl
from jax.experimental.pallas import tpu as pltpu
from jax.experimental.pallas import tpu_sc as plsc

NC = 2
NS = 16
L = 16
RW = 128
NW = NC * NS


def _geometry(n, m):
    rows = -(-n // (NW * RW))
    m_pad = (m // (NS * L) + 1) * NS * L
    return rows, m_pad


def _make_k1(n, m, rows, m_pad):
    msl = m_pad // NS
    ch = rows * RW
    v_last = n - (NW - 1) * ch
    tail = ch - v_last
    pr = ((m_pad - m) // L) * L

    def body(idx_hbm, chg_hbm, psum_hbm, pcnt_hbm,
             ssum, scnt, idx_v, val_v, zf, zi, cpos_v,
             sysr, cntr, sumr, sem_st):
        c = lax.axis_index("c")
        s = lax.axis_index("s")
        wid = s * NC + c
        base = wid * ch

        @pl.when(wid < NW - 1)
        def _():
            pltpu.async_copy(idx_hbm.at[pl.ds(base, ch)],
                             idx_v.at[pl.ds(0, ch)], sem_st)
            pltpu.async_copy(chg_hbm.at[pl.ds(base, ch)], val_v, sem_st)

        @pl.when(wid == NW - 1)
        def _():
            pltpu.async_copy(idx_hbm.at[pl.ds(base, v_last)],
                             idx_v.at[pl.ds(0, v_last)], sem_st)
            pltpu.async_copy(chg_hbm.at[pl.ds(base, v_last)],
                             val_v.at[pl.ds(0, v_last)], sem_st)

        def zbody(i, carry):
            zf[pl.ds(i * L, L)] = jnp.zeros((L,), jnp.float32)
            zi[pl.ds(i * L, L)] = jnp.zeros((L,), jnp.int32)
            return carry

        lax.fori_loop(0, msl // L, zbody, 0)
        pltpu.sync_copy(zf, ssum.at[pl.ds(s * msl, msl)])
        pltpu.sync_copy(zi, scnt.at[pl.ds(s * msl, msl)])
        plsc.subcore_barrier()

        @pl.when(wid < NW - 1)
        def _():
            pltpu.make_async_copy(idx_hbm.at[pl.ds(base, ch)],
                                  idx_v.at[pl.ds(0, ch)], sem_st).wait()
            pltpu.make_async_copy(chg_hbm.at[pl.ds(base, ch)],
                                  val_v, sem_st).wait()

        @pl.when(wid == NW - 1)
        def _():
            pltpu.make_async_copy(idx_hbm.at[pl.ds(base, v_last)],
                                  idx_v.at[pl.ds(0, v_last)], sem_st).wait()
            pltpu.make_async_copy(chg_hbm.at[pl.ds(base, v_last)],
                                  val_v.at[pl.ds(0, v_last)], sem_st).wait()

            def tbody(i, carry):
                lanes = pl.ds(v_last + i * L, L)
                idx_v[lanes] = jnp.full((L,), m, jnp.int32) + lax.rem(
                    i, jnp.int32(pr))
                val_v[lanes] = jnp.zeros((L,), jnp.float32)
                return carry

            lax.fori_loop(0, tail // L, tbody, 0)

        ve = jnp.where(wid == NW - 1, v_last, ch)
        idx_v[pl.ds(ve, L)] = jnp.full((L,), m, jnp.int32)
        cpos_v[pl.ds(0, L)] = jnp.zeros((L,), jnp.int32)
        ptr0 = jnp.int32(1)
        iota_l = lax.iota(jnp.int32, L)

        def kbody(i, carry):
            ptr, cy = carry
            sl = pl.ds(i * L, L)
            pos = jnp.full((L,), i * L, jnp.int32) + iota_l
            idxv = idx_v[sl]
            sh = plsc.load_gather(idx_v, [jnp.maximum(pos - 1, 0)])
            mask = idxv != sh
            cum = plsc.cumsum(val_v[sl]) + cy
            val_v[sl] = cum
            cy2 = plsc.load_gather(
                val_v, [jnp.full((L,), i * L + L - 1, jnp.int32)])
            plsc.store_compressed(cpos_v.at[pl.ds(ptr, L)], pos, mask=mask)
            pc = plsc.all_reduce_population_count(mask)
            return ptr + jnp.max(pc), cy2

        nb, _ = lax.fori_loop(0, rows * (RW // L), kbody,
                              (ptr0, jnp.zeros((L,), jnp.float32)))

        def pbody(k, carry):
            cpos_v[pl.ds(nb + k * L, L)] = jnp.full((L,), ve, jnp.int32)
            return carry

        lax.fori_loop(0, (RW + L) // L + 1, pbody, 0)

        def rbody(r, carry):
            for q in range(RW // L):
                qpos = jnp.full((L,), 0, jnp.int32) + r * RW + q * L + iota_l
                cp = plsc.load_gather(cpos_v, [qpos])
                cpn = plsc.load_gather(cpos_v, [qpos + 1])
                sysr[pl.ds(q * L, L)] = plsc.load_gather(idx_v, [cp])
                cntr[pl.ds(q * L, L)] = cpn - cp
                cprev = plsc.load_gather(val_v, [jnp.maximum(cp - 1, 0)])
                cprev = jnp.where(cp == 0, jnp.zeros((L,), jnp.float32),
                                  cprev)
                cend = plsc.load_gather(val_v, [jnp.maximum(cpn - 1, 0)])
                sumr[pl.ds(q * L, L)] = cend - cprev
            pltpu.sync_copy(sumr, ssum.at[sysr], add=True)
            pltpu.sync_copy(cntr, scnt.at[sysr], add=True)
            return carry

        lax.fori_loop(0, (nb + RW - 1) // RW, rbody, 0)
        plsc.subcore_barrier()
        pltpu.sync_copy(ssum.at[pl.ds(s * msl, msl)], zf)
        pltpu.sync_copy(zf, psum_hbm.at[pl.ds(c * m_pad + s * msl, msl)])
        pltpu.sync_copy(scnt.at[pl.ds(s * msl, msl)], zi)
        pltpu.sync_copy(zi, pcnt_hbm.at[pl.ds(c * m_pad + s * msl, msl)])

    return pl.kernel(
        body,
        out_type=[jax.ShapeDtypeStruct((NC * m_pad,), jnp.float32),
                  jax.ShapeDtypeStruct((NC * m_pad,), jnp.int32)],
        mesh=plsc.VectorSubcoreMesh(core_axis_name="c", subcore_axis_name="s",
                                    num_cores=NC, num_subcores=NS),
        compiler_params=pltpu.CompilerParams(needs_layout_passes=False),
        scratch_types=[
            pltpu.VMEM_SHARED((m_pad,), jnp.float32),
            pltpu.VMEM_SHARED((m_pad,), jnp.int32),
            pltpu.VMEM((ch + L,), jnp.int32),
            pltpu.VMEM((ch,), jnp.float32),
            pltpu.VMEM((msl,), jnp.float32),
            pltpu.VMEM((msl,), jnp.int32),
            pltpu.VMEM((ch + 2 * RW,), jnp.int32),
            pltpu.VMEM((RW,), jnp.int32),
            pltpu.VMEM((RW,), jnp.int32),
            pltpu.VMEM((RW,), jnp.float32),
            pltpu.SemaphoreType.DMA,
        ],
    )


def _make_k2(n, m, rows, m_pad):
    msl = m_pad // NS
    ch = rows * RW
    v_last = n - (NW - 1) * ch
    tail = ch - v_last
    pr = ((m_pad - m) // L) * L
    m_tail = m - (NS - 1) * msl
    CW = 1024

    def body(idx_hbm, chg_hbm, psum_hbm, pcnt_hbm, tot_hbm, out_hbm,
             scorr, idx_v, val_v, win_v, s0v, s1v, n0v, n1v, totv, cbuf,
             sem_st):
        c = lax.axis_index("c")
        s = lax.axis_index("s")
        wid = s * NC + c
        base = wid * ch
        bm = s * msl

        @pl.when(wid < NW - 1)
        def _():
            pltpu.async_copy(idx_hbm.at[pl.ds(base, ch)], idx_v, sem_st)
            pltpu.async_copy(chg_hbm.at[pl.ds(base, ch)], val_v, sem_st)

        @pl.when(wid == NW - 1)
        def _():
            pltpu.async_copy(idx_hbm.at[pl.ds(base, v_last)],
                             idx_v.at[pl.ds(0, v_last)], sem_st)
            pltpu.async_copy(chg_hbm.at[pl.ds(base, v_last)],
                             val_v.at[pl.ds(0, v_last)], sem_st)

        pltpu.sync_copy(psum_hbm.at[pl.ds(bm, msl)], s0v)
        pltpu.sync_copy(psum_hbm.at[pl.ds(m_pad + bm, msl)], s1v)
        pltpu.sync_copy(pcnt_hbm.at[pl.ds(bm, msl)], n0v)
        pltpu.sync_copy(pcnt_hbm.at[pl.ds(m_pad + bm, msl)], n1v)

        @pl.when(s < NS - 1)
        def _():
            pltpu.sync_copy(tot_hbm.at[pl.ds(bm, msl)], totv)

        @pl.when(s == NS - 1)
        def _():
            pltpu.sync_copy(tot_hbm.at[pl.ds(bm, m_tail)],
                            totv.at[pl.ds(0, m_tail)])

        def cbody(i, carry):
            sl = pl.ds(i * L, L)
            seg_sum = s0v[sl] + s1v[sl]
            cnt = (n0v[sl] + n1v[sl]).astype(jnp.float32)
            cbuf[sl] = (totv[sl] - seg_sum) / cnt
            return carry

        lax.fori_loop(0, msl // L, cbody, 0)
        pltpu.sync_copy(cbuf, scorr.at[pl.ds(bm, msl)])
        plsc.subcore_barrier()

        @pl.when(wid < NW - 1)
        def _():
            pltpu.make_async_copy(idx_hbm.at[pl.ds(base, ch)],
                                  idx_v, sem_st).wait()
            pltpu.make_async_copy(chg_hbm.at[pl.ds(base, ch)],
                                  val_v, sem_st).wait()

        @pl.when(wid == NW - 1)
        def _():
            pltpu.make_async_copy(idx_hbm.at[pl.ds(base, v_last)],
                                  idx_v.at[pl.ds(0, v_last)], sem_st).wait()
            pltpu.make_async_copy(chg_hbm.at[pl.ds(base, v_last)],
                                  val_v.at[pl.ds(0, v_last)], sem_st).wait()

            def tbody(i, carry):
                lanes = pl.ds(v_last + i * L, L)
                idx_v[lanes] = jnp.full((L,), m, jnp.int32) + lax.rem(
                    i, jnp.int32(pr))
                val_v[lanes] = jnp.zeros((L,), jnp.float32)
                return carry

            lax.fori_loop(0, tail // L, tbody, 0)

        ve = jnp.where(wid == NW - 1, v_last, ch)
        lo8 = (idx_v[pl.ds(0, L)][0] // 8) * 8
        hi = idx_v[pl.ds(ve - L, L)][L - 1]
        width = hi - lo8 + 1
        ncw = (width + CW - 1) // CW
        start_cap = ((m_pad - CW) // 8) * 8

        def wbody(k, carry):
            st = jnp.minimum(lo8 + k * CW, start_cap)
            pltpu.sync_copy(scorr.at[pl.ds(st, CW)],
                            win_v.at[pl.ds(st - lo8, CW)])
            return carry

        lax.fori_loop(0, ncw, wbody, 0)

        lo_v = jnp.full((L,), lo8, jnp.int32)

        def gbody(j, carry):
            for l in range(RW // L):
                sl = pl.ds(j * RW + l * L, L)
                off = idx_v[sl] - lo_v
                g = plsc.load_gather(win_v, [off])
                val_v[sl] = val_v[sl] + g
            return carry

        lax.fori_loop(0, rows, gbody, 0)

        @pl.when(wid < NW - 1)
        def _():
            pltpu.sync_copy(val_v, out_hbm.at[pl.ds(base, ch)])

        @pl.when(wid == NW - 1)
        def _():
            pltpu.sync_copy(val_v.at[pl.ds(0, v_last)],
                            out_hbm.at[pl.ds(base, v_last)])

    return pl.kernel(
        body,
        out_type=jax.ShapeDtypeStruct((n,), jnp.float32),
        mesh=plsc.VectorSubcoreMesh(core_axis_name="c", subcore_axis_name="s",
                                    num_cores=NC, num_subcores=NS),
        compiler_params=pltpu.CompilerParams(needs_layout_passes=False),
        scratch_types=[
            pltpu.VMEM_SHARED((m_pad,), jnp.float32),
            pltpu.VMEM((ch,), jnp.int32),
            pltpu.VMEM((ch,), jnp.float32),
            pltpu.VMEM((m_pad,), jnp.float32),
            pltpu.VMEM((msl,), jnp.float32),
            pltpu.VMEM((msl,), jnp.float32),
            pltpu.VMEM((msl,), jnp.int32),
            pltpu.VMEM((msl,), jnp.int32),
            pltpu.VMEM((msl,), jnp.float32),
            pltpu.VMEM((msl,), jnp.float32),
            pltpu.SemaphoreType.DMA,
        ],
    )


def kernel(per_atom_charge, per_system_total_charge, atomic_subsystem_indices):
    n = per_atom_charge.shape[0]
    m = per_system_total_charge.shape[0]
    rows, m_pad = _geometry(n, m)

    chg = per_atom_charge.reshape(-1).astype(jnp.float32)
    idx = atomic_subsystem_indices.astype(jnp.int32)
    tot = per_system_total_charge.reshape(-1).astype(jnp.float32)

    psum, pcnt = _make_k1(n, m, rows, m_pad)(idx, chg)
    out = _make_k2(n, m, rows, m_pad)(idx, chg, psum, pcnt, tot)
    return out[:, None]
